# Initial kernel scaffold; baseline (speedup 1.0000x reference)
import jax
import jax.numpy as jnp
from jax.experimental import pallas as pl


def kernel(data, ids, space_pts, time_pts, query_pts, te_w, te_phase, s0_Wf, s0_Wr, s0_b, t0_Wf, t0_Wr, t0_b, c0_W1, c0_b1, c0_W2, c0_b2, s1_Wf, s1_Wr, s1_b, t1_Wf, t1_Wr, t1_b, c1_W1, c1_b1, c1_W2, c1_b2, tg_Wf, tg_Wr, tg_b):
    raise NotImplementedError("write your pallas kernel here")


# speedup vs baseline: 2.1991x; 2.1991x over previous
import functools

import jax
import jax.numpy as jnp
from jax.experimental import pallas as pl

NEIGHBORS, TIMESTEPS = 16, 8


def _knn_body(qpts_ref, kpts_ref, idx_ref, rel_ref, *, K, D, N, want_rel):
    d2 = None
    diff0 = None
    for d in range(D):
        qcol = qpts_ref[0, :, d:d + 1]
        krow = kpts_ref[0, d:d + 1, :]
        diff = qcol - krow
        sq = diff * diff
        d2 = sq if d2 is None else d2 + sq
        if want_rel and d == 0:
            diff0 = diff
    iota = jax.lax.broadcasted_iota(jnp.int32, (1, N), 1)
    big = jnp.int32(N)
    for k in range(K):
        m = jnp.min(d2, axis=1, keepdims=True)
        am = jnp.min(jnp.where(d2 == m, iota, big), axis=1,
                     keepdims=True)
        idx_ref[0, :, k:k + 1] = am
        hit = iota == am
        if want_rel:
            rel_ref[0, :, k:k + 1] = jnp.sum(
                jnp.where(hit, diff0, 0.0), axis=1, keepdims=True)
        d2 = jnp.where(hit, jnp.inf, d2)


def _knn(qpts, kpts_t, K, want_rel, tq):
    B, Nq, D = qpts.shape
    N = kpts_t.shape[2]
    grid = (B, Nq // tq)
    out_shapes = [jax.ShapeDtypeStruct((B, Nq, K), jnp.int32)]
    out_specs = [pl.BlockSpec((1, tq, K), lambda b, i: (b, i, 0))]
    if want_rel:
        out_shapes.append(jax.ShapeDtypeStruct((B, Nq, K), jnp.float32))
        out_specs.append(pl.BlockSpec((1, tq, K), lambda b, i: (b, i, 0)))
    body = functools.partial(_knn_body, K=K, D=D, N=N, want_rel=want_rel)
    if not want_rel:
        body2 = lambda q, kk, i: body(q, kk, i, None)
    else:
        body2 = body
    res = pl.pallas_call(
        body2,
        grid=grid,
        in_specs=[
            pl.BlockSpec((1, tq, D), lambda b, i: (b, i, 0)),
            pl.BlockSpec((1, D, N), lambda b, i: (b, 0, 0)),
        ],
        out_specs=out_specs,
        out_shape=out_shapes,
    )(qpts, kpts_t)
    return res if want_rel else (res[0], None)


def _gmean(z, idx, add_pq=None, add_pqk=None):
    g = jnp.take_along_axis(z[:, None, :, :], idx[..., None], axis=2)
    h = g
    if add_pq is not None:
        h = h + add_pq[:, :, None, :]
    if add_pqk is not None:
        h = h + add_pqk
    return jnp.mean(jax.nn.relu(h), axis=2)


def kernel(data, ids, space_pts, time_pts, query_pts, te_w, te_phase,
           s0_Wf, s0_Wr, s0_b, t0_Wf, t0_Wr, t0_b, c0_W1, c0_b1, c0_W2, c0_b2,
           s1_Wf, s1_Wr, s1_b, t1_Wf, t1_Wr, t1_b, c1_W1, c1_b1, c1_W2, c1_b2,
           tg_Wf, tg_Wr, tg_b):
    B, N, F = data.shape
    Q = query_pts.shape[1]

    sp_t = jnp.transpose(space_pts, (0, 2, 1))
    tp_t = jnp.transpose(time_pts, (0, 2, 1))

    idx_s, _ = _knn(space_pts, sp_t, NEIGHBORS, False, 256)
    idx_t, rel_t = _knn(time_pts, tp_t, TIMESTEPS, True, 256)
    idx_q, rel_q = _knn(query_pts, tp_t, TIMESTEPS, True, 512)

    E_t = jnp.cos(rel_t[..., None] * te_w + te_phase)
    E_q = jnp.cos(rel_q[..., None] * te_w + te_phase)

    x = data
    for (sWf, sWr, sb, tWf, tWr, tb, cW1, cb1, cW2, cb2) in [
            (s0_Wf, s0_Wr, s0_b, t0_Wf, t0_Wr, t0_b, c0_W1, c0_b1, c0_W2, c0_b2),
            (s1_Wf, s1_Wr, s1_b, t1_Wf, t1_Wr, t1_b, c1_W1, c1_b1, c1_W2, c1_b2)]:
        u = space_pts @ sWr
        z_s = x @ sWf - u
        a_s = u + sb
        snei = _gmean(z_s, idx_s, add_pq=a_s)
        z_t = x @ tWf[:F] + snei @ tWf[F:]
        relc = E_t @ tWr + tb
        tnei = _gmean(z_t, idx_t, add_pqk=relc)
        x = jnp.maximum(
            x @ cW1[:F] + snei @ cW1[F:F + 64] + tnei @ cW1[F + 64:] + cb1,
            0.0) @ cW2 + cb2
    z_g = x @ tg_Wf
    relc_q = E_q @ tg_Wr + tg_b
    return _gmean(z_g, idx_q, add_pqk=relc_q)


# speedup vs baseline: 15.5314x; 7.0628x over previous
import functools

import jax
import jax.numpy as jnp
from jax import lax
from jax.experimental import pallas as pl
from jax.experimental.pallas import tpu as pltpu
from jax.experimental.pallas import tpu_sc as plsc

NEIGHBORS, TIMESTEPS = 16, 8
_NW = 32


def _knn_body(qpts_ref, kpts_ref, idx_ref, rel_ref, *, K, D, N, want_rel):
    d2 = None
    diff0 = None
    for d in range(D):
        qcol = qpts_ref[0, :, d:d + 1]
        krow = kpts_ref[0, d:d + 1, :]
        diff = qcol - krow
        sq = diff * diff
        d2 = sq if d2 is None else d2 + sq
        if want_rel and d == 0:
            diff0 = diff
    iota = jax.lax.broadcasted_iota(jnp.int32, (1, N), 1)
    big = jnp.int32(N)
    for k in range(K):
        m = jnp.min(d2, axis=1, keepdims=True)
        am = jnp.min(jnp.where(d2 == m, iota, big), axis=1,
                     keepdims=True)
        idx_ref[0, :, k:k + 1] = am
        hit = iota == am
        if want_rel:
            rel_ref[0, :, k:k + 1] = jnp.sum(
                jnp.where(hit, diff0, 0.0), axis=1, keepdims=True)
        d2 = jnp.where(hit, jnp.inf, d2)


def _knn(qpts, kpts_t, K, want_rel, tq):
    B, Nq, D = qpts.shape
    N = kpts_t.shape[2]
    grid = (B, Nq // tq)
    out_shapes = [jax.ShapeDtypeStruct((B, Nq, K), jnp.int32)]
    out_specs = [pl.BlockSpec((1, tq, K), lambda b, i: (b, i, 0))]
    if want_rel:
        out_shapes.append(jax.ShapeDtypeStruct((B, Nq, K), jnp.float32))
        out_specs.append(pl.BlockSpec((1, tq, K), lambda b, i: (b, i, 0)))
    body = functools.partial(_knn_body, K=K, D=D, N=N, want_rel=want_rel)
    if not want_rel:
        body2 = lambda q, kk, i: body(q, kk, i, None)
    else:
        body2 = body
    res = pl.pallas_call(
        body2,
        grid=grid,
        in_specs=[
            pl.BlockSpec((1, tq, D), lambda b, i: (b, i, 0)),
            pl.BlockSpec((1, D, N), lambda b, i: (b, 0, 0)),
        ],
        out_specs=out_specs,
        out_shape=out_shapes,
    )(qpts, kpts_t)
    return res if want_rel else (res[0], None)


@functools.lru_cache(maxsize=None)
def _gmean_sc(NQ, K, C, per_qk, CQ):
    nq_w = NQ // _NW
    steps = nq_w // CQ
    assert CQ * K <= 128 and nq_w % CQ == 0
    nj = C // 16
    mesh = plsc.VectorSubcoreMesh(core_axis_name="c", subcore_axis_name="s")

    @functools.partial(
        pl.kernel, mesh=mesh,
        out_type=jax.ShapeDtypeStruct((NQ, C), jnp.float32),
        scratch_types=[
            pltpu.VMEM((CQ * K,), jnp.int32),
            pltpu.VMEM((CQ * K, 128), jnp.float32),
            pltpu.VMEM((CQ * K if per_qk else CQ, C), jnp.float32),
            pltpu.VMEM((CQ, C), jnp.float32),
            pltpu.SemaphoreType.DMA,
        ])
    def kfun(z_hbm, gidx_hbm, add_hbm, out_hbm, idx_v, rows_v, add_v, out_v, sem):
        wid = lax.axis_index("s") * 2 + lax.axis_index("c")
        qbase = wid * nq_w

        def step(s, carry):
            qb = qbase + s * CQ
            pltpu.sync_copy(gidx_hbm.at[pl.ds(qb * K, CQ * K)], idx_v)
            if per_qk:
                pltpu.sync_copy(add_hbm.at[pl.ds(qb * K, CQ * K)], add_v)
            else:
                pltpu.sync_copy(add_hbm.at[pl.ds(qb, CQ)], add_v)
            pltpu.async_copy(z_hbm.at[idx_v], rows_v, sem).wait()
            inv_k = jnp.float32(1.0 / K)
            for q in range(CQ):
                for j in range(nj):
                    sl = pl.ds(j * 16, 16)
                    acc = jnp.zeros((16,), jnp.float32)
                    if not per_qk:
                        a = add_v[q, sl]
                    for k in range(K):
                        r = q * K + k
                        h = rows_v[r, sl] + (add_v[r, sl] if per_qk else a)
                        acc = acc + jnp.maximum(h, 0.0)
                    out_v[q, sl] = acc * inv_k
            pltpu.sync_copy(out_v, out_hbm.at[pl.ds(qb, CQ)])
            return carry

        lax.fori_loop(0, steps, step, 0)

    return kfun


def _gmean(z, idx, add_pq=None, add_pqk=None):
    B, NR, C = z.shape
    _, NQ_b, K = idx.shape
    NQ = B * NQ_b
    gidx = (idx + (jnp.arange(B, dtype=jnp.int32) * NR)[:, None, None])
    gidx = gidx.reshape(NQ * K)
    zf = z.reshape(B * NR, C)
    if C < 128:
        zf = jnp.pad(zf, ((0, 0), (0, 128 - C)))
    if add_pqk is not None:
        add = add_pqk.reshape(NQ * K, C)
        per_qk = True
    else:
        add = add_pq.reshape(NQ, C)
        per_qk = False
    CQ = min(128 // K, 16 if C <= 64 else 8)
    out = _gmean_sc(NQ, K, C, per_qk, CQ)(zf, gidx, add)
    return out.reshape(B, NQ_b, C)


def kernel(data, ids, space_pts, time_pts, query_pts, te_w, te_phase,
           s0_Wf, s0_Wr, s0_b, t0_Wf, t0_Wr, t0_b, c0_W1, c0_b1, c0_W2, c0_b2,
           s1_Wf, s1_Wr, s1_b, t1_Wf, t1_Wr, t1_b, c1_W1, c1_b1, c1_W2, c1_b2,
           tg_Wf, tg_Wr, tg_b):
    B, N, F = data.shape
    Q = query_pts.shape[1]

    sp_t = jnp.transpose(space_pts, (0, 2, 1))
    tp_t = jnp.transpose(time_pts, (0, 2, 1))

    idx_s, _ = _knn(space_pts, sp_t, NEIGHBORS, False, 256)
    idx_t, rel_t = _knn(time_pts, tp_t, TIMESTEPS, True, 256)
    idx_q, rel_q = _knn(query_pts, tp_t, TIMESTEPS, True, 512)

    E_t = jnp.cos(rel_t[..., None] * te_w + te_phase)
    E_q = jnp.cos(rel_q[..., None] * te_w + te_phase)

    x = data
    for (sWf, sWr, sb, tWf, tWr, tb, cW1, cb1, cW2, cb2) in [
            (s0_Wf, s0_Wr, s0_b, t0_Wf, t0_Wr, t0_b, c0_W1, c0_b1, c0_W2, c0_b2),
            (s1_Wf, s1_Wr, s1_b, t1_Wf, t1_Wr, t1_b, c1_W1, c1_b1, c1_W2, c1_b2)]:
        u = space_pts @ sWr
        z_s = x @ sWf - u
        a_s = u + sb
        snei = _gmean(z_s, idx_s, add_pq=a_s)
        z_t = x @ tWf[:F] + snei @ tWf[F:]
        relc = E_t @ tWr + tb
        tnei = _gmean(z_t, idx_t, add_pqk=relc)
        x = jnp.maximum(
            x @ cW1[:F] + snei @ cW1[F:F + 64] + tnei @ cW1[F + 64:] + cb1,
            0.0) @ cW2 + cb2
    z_g = x @ tg_Wf
    relc_q = E_q @ tg_Wr + tg_b
    return _gmean(z_g, idx_q, add_pqk=relc_q)


# speedup vs baseline: 17.0599x; 1.0984x over previous
import functools

import jax
import jax.numpy as jnp
from jax import lax
from jax.experimental import pallas as pl
from jax.experimental.pallas import tpu as pltpu
from jax.experimental.pallas import tpu_sc as plsc

NEIGHBORS, TIMESTEPS = 16, 8
_NW = 32


def _knn_body(qpts_ref, kpts_ref, idx_ref, rel_ref, *, K, D, N, want_rel):
    d2 = None
    diff0 = None
    for d in range(D):
        qcol = qpts_ref[0, :, d:d + 1]
        krow = kpts_ref[0, d:d + 1, :]
        diff = qcol - krow
        sq = diff * diff
        d2 = sq if d2 is None else d2 + sq
        if want_rel and d == 0:
            diff0 = diff
    iota = jax.lax.broadcasted_iota(jnp.int32, (1, N), 1)
    big = jnp.int32(N)
    for k in range(K):
        m = jnp.min(d2, axis=1, keepdims=True)
        am = jnp.min(jnp.where(d2 == m, iota, big), axis=1,
                     keepdims=True)
        idx_ref[0, :, k:k + 1] = am
        hit = iota == am
        if want_rel:
            rel_ref[0, :, k:k + 1] = jnp.sum(
                jnp.where(hit, diff0, 0.0), axis=1, keepdims=True)
        d2 = jnp.where(hit, jnp.inf, d2)


def _knn(qpts, kpts_t, K, want_rel, tq):
    B, Nq, D = qpts.shape
    N = kpts_t.shape[2]
    grid = (B, Nq // tq)
    out_shapes = [jax.ShapeDtypeStruct((B, Nq, K), jnp.int32)]
    out_specs = [pl.BlockSpec((1, tq, K), lambda b, i: (b, i, 0))]
    if want_rel:
        out_shapes.append(jax.ShapeDtypeStruct((B, Nq, K), jnp.float32))
        out_specs.append(pl.BlockSpec((1, tq, K), lambda b, i: (b, i, 0)))
    body = functools.partial(_knn_body, K=K, D=D, N=N, want_rel=want_rel)
    if not want_rel:
        body2 = lambda q, kk, i: body(q, kk, i, None)
    else:
        body2 = body
    res = pl.pallas_call(
        body2,
        grid=grid,
        in_specs=[
            pl.BlockSpec((1, tq, D), lambda b, i: (b, i, 0)),
            pl.BlockSpec((1, D, N), lambda b, i: (b, 0, 0)),
        ],
        out_specs=out_specs,
        out_shape=out_shapes,
    )(qpts, kpts_t)
    return res if want_rel else (res[0], None)


@functools.lru_cache(maxsize=None)
def _gmean_sc_v2(NQ, K, C, per_qk, CQ):
    nq_w = NQ // _NW
    steps = nq_w // CQ
    assert CQ * K <= 128 and steps % 2 == 0
    nj = C // 16
    nk_rows = CQ * K
    mesh = plsc.VectorSubcoreMesh(core_axis_name="c", subcore_axis_name="s")
    add_rows = nk_rows if per_qk else nq_w

    @functools.partial(
        pl.kernel, mesh=mesh,
        out_type=jax.ShapeDtypeStruct((NQ, C), jnp.float32),
        scratch_types=[
            pltpu.VMEM((nq_w * K,), jnp.int32),
            pltpu.VMEM((nk_rows, 128), jnp.float32),
            pltpu.VMEM((nk_rows, 128), jnp.float32),
            pltpu.VMEM((add_rows, C), jnp.float32),
            pltpu.VMEM((nk_rows if per_qk else 1, C), jnp.float32),
            pltpu.VMEM((nq_w, C), jnp.float32),
            pltpu.SemaphoreType.DMA,
            pltpu.SemaphoreType.DMA,
        ])
    def kfun(z_hbm, gidx_hbm, add_hbm, out_hbm,
             idx_all, rows0, rows1, add0, add1, out_v, sem0, sem1):
        wid = lax.axis_index("s") * 2 + lax.axis_index("c")
        qbase = wid * nq_w
        rows_b = (rows0, rows1)
        add_b = (add0, add1)
        sem_b = (sem0, sem1)

        pltpu.sync_copy(gidx_hbm.at[pl.ds(qbase * K, nq_w * K)], idx_all)
        if not per_qk:
            pltpu.sync_copy(add_hbm.at[pl.ds(qbase, nq_w)], add0)

        def fire(c, b):
            pltpu.async_copy(
                z_hbm.at[idx_all.at[pl.ds(c * nk_rows, nk_rows)]],
                rows_b[b], sem_b[b])
            if per_qk:
                pltpu.async_copy(
                    add_hbm.at[pl.ds((qbase + c * CQ) * K, nk_rows)],
                    add_b[b], sem_b[b])

        def drain(b):
            pltpu.make_async_copy(
                z_hbm.at[pl.ds(0, nk_rows)], rows_b[b], sem_b[b]).wait()
            if per_qk:
                pltpu.make_async_copy(
                    add_hbm.at[pl.ds(0, nk_rows)], add_b[b], sem_b[b]).wait()

        def compute(c, b):
            rows = rows_b[b]
            adds = add_b[b]
            qloc = c * CQ
            inv_k = jnp.float32(1.0 / K)
            for q in range(CQ):
                for j in range(nj):
                    sl = pl.ds(j * 16, 16)
                    acc = jnp.zeros((16,), jnp.float32)
                    if not per_qk:
                        a = add0[qloc + q, sl]
                    for k in range(K):
                        r = q * K + k
                        h = rows[r, sl] + (adds[r, sl] if per_qk else a)
                        acc = acc + jnp.maximum(h, 0.0)
                    out_v[qloc + q, sl] = acc * inv_k

        fire(0, 0)

        def step(s, carry):
            c0 = 2 * s
            fire(c0 + 1, 1)
            drain(0)
            compute(c0, 0)
            fire(jnp.minimum(c0 + 2, steps - 1), 0)
            drain(1)
            compute(c0 + 1, 1)
            return carry

        lax.fori_loop(0, steps // 2, step, 0)
        drain(0)
        pltpu.sync_copy(out_v, out_hbm.at[pl.ds(qbase, nq_w)])

    return kfun


@functools.lru_cache(maxsize=None)
def _gmean_sc(NQ, K, C, per_qk, CQ):
    nq_w = NQ // _NW
    steps = nq_w // CQ
    assert CQ * K <= 128 and nq_w % CQ == 0
    nj = C // 16
    mesh = plsc.VectorSubcoreMesh(core_axis_name="c", subcore_axis_name="s")

    @functools.partial(
        pl.kernel, mesh=mesh,
        out_type=jax.ShapeDtypeStruct((NQ, C), jnp.float32),
        scratch_types=[
            pltpu.VMEM((CQ * K,), jnp.int32),
            pltpu.VMEM((CQ * K, 128), jnp.float32),
            pltpu.VMEM((CQ * K if per_qk else CQ, C), jnp.float32),
            pltpu.VMEM((CQ, C), jnp.float32),
            pltpu.SemaphoreType.DMA,
        ])
    def kfun(z_hbm, gidx_hbm, add_hbm, out_hbm, idx_v, rows_v, add_v, out_v, sem):
        wid = lax.axis_index("s") * 2 + lax.axis_index("c")
        qbase = wid * nq_w

        def step(s, carry):
            qb = qbase + s * CQ
            pltpu.sync_copy(gidx_hbm.at[pl.ds(qb * K, CQ * K)], idx_v)
            if per_qk:
                pltpu.sync_copy(add_hbm.at[pl.ds(qb * K, CQ * K)], add_v)
            else:
                pltpu.sync_copy(add_hbm.at[pl.ds(qb, CQ)], add_v)
            pltpu.async_copy(z_hbm.at[idx_v], rows_v, sem).wait()
            inv_k = jnp.float32(1.0 / K)
            for q in range(CQ):
                for j in range(nj):
                    sl = pl.ds(j * 16, 16)
                    acc = jnp.zeros((16,), jnp.float32)
                    if not per_qk:
                        a = add_v[q, sl]
                    for k in range(K):
                        r = q * K + k
                        h = rows_v[r, sl] + (add_v[r, sl] if per_qk else a)
                        acc = acc + jnp.maximum(h, 0.0)
                    out_v[q, sl] = acc * inv_k
            pltpu.sync_copy(out_v, out_hbm.at[pl.ds(qb, CQ)])
            return carry

        lax.fori_loop(0, steps, step, 0)

    return kfun


def _gmean(z, idx, add_pq=None, add_pqk=None):
    B, NR, C = z.shape
    _, NQ_b, K = idx.shape
    NQ = B * NQ_b
    gidx = (idx + (jnp.arange(B, dtype=jnp.int32) * NR)[:, None, None])
    gidx = gidx.reshape(NQ * K)
    zf = z.reshape(B * NR, C)
    if C < 128:
        zf = jnp.pad(zf, ((0, 0), (0, 128 - C)))
    if add_pqk is not None:
        add = add_pqk.reshape(NQ * K, C)
        per_qk = True
    else:
        add = add_pq.reshape(NQ, C)
        per_qk = False
    CQ = min(128 // K, 16 if C <= 64 else 8)
    out = _gmean_sc_v2(NQ, K, C, per_qk, CQ)(zf, gidx, add)
    return out.reshape(B, NQ_b, C)


def kernel(data, ids, space_pts, time_pts, query_pts, te_w, te_phase,
           s0_Wf, s0_Wr, s0_b, t0_Wf, t0_Wr, t0_b, c0_W1, c0_b1, c0_W2, c0_b2,
           s1_Wf, s1_Wr, s1_b, t1_Wf, t1_Wr, t1_b, c1_W1, c1_b1, c1_W2, c1_b2,
           tg_Wf, tg_Wr, tg_b):
    B, N, F = data.shape
    Q = query_pts.shape[1]

    sp_t = jnp.transpose(space_pts, (0, 2, 1))
    tp_t = jnp.transpose(time_pts, (0, 2, 1))

    idx_s, _ = _knn(space_pts, sp_t, NEIGHBORS, False, 256)
    idx_t, rel_t = _knn(time_pts, tp_t, TIMESTEPS, True, 256)
    idx_q, rel_q = _knn(query_pts, tp_t, TIMESTEPS, True, 512)

    E_t = jnp.cos(rel_t[..., None] * te_w + te_phase)
    E_q = jnp.cos(rel_q[..., None] * te_w + te_phase)

    x = data
    for (sWf, sWr, sb, tWf, tWr, tb, cW1, cb1, cW2, cb2) in [
            (s0_Wf, s0_Wr, s0_b, t0_Wf, t0_Wr, t0_b, c0_W1, c0_b1, c0_W2, c0_b2),
            (s1_Wf, s1_Wr, s1_b, t1_Wf, t1_Wr, t1_b, c1_W1, c1_b1, c1_W2, c1_b2)]:
        u = space_pts @ sWr
        z_s = x @ sWf - u
        a_s = u + sb
        snei = _gmean(z_s, idx_s, add_pq=a_s)
        z_t = x @ tWf[:F] + snei @ tWf[F:]
        relc = E_t @ tWr + tb
        tnei = _gmean(z_t, idx_t, add_pqk=relc)
        x = jnp.maximum(
            x @ cW1[:F] + snei @ cW1[F:F + 64] + tnei @ cW1[F + 64:] + cb1,
            0.0) @ cW2 + cb2
    z_g = x @ tg_Wf
    relc_q = E_q @ tg_Wr + tg_b
    return _gmean(z_g, idx_q, add_pqk=relc_q)


# speedup vs baseline: 18.4399x; 1.0809x over previous
import functools

import jax
import jax.numpy as jnp
from jax import lax
from jax.experimental import pallas as pl
from jax.experimental.pallas import tpu as pltpu
from jax.experimental.pallas import tpu_sc as plsc

NEIGHBORS, TIMESTEPS = 16, 8
_NW = 32


def _knn_body(qpts_ref, kpts_ref, idx_ref, rel_ref, *, K, D, N, want_rel):
    d2 = None
    diff0 = None
    for d in range(D):
        qcol = qpts_ref[0, :, d:d + 1]
        krow = kpts_ref[0, d:d + 1, :]
        diff = qcol - krow
        sq = diff * diff
        d2 = sq if d2 is None else d2 + sq
        if want_rel and d == 0:
            diff0 = diff
    iota = jax.lax.broadcasted_iota(jnp.int32, (1, N), 1)
    big = jnp.int32(N)
    for k in range(K):
        if want_rel:
            m = jnp.min(d2, axis=1, keepdims=True)
            am = jnp.min(jnp.where(d2 == m, iota, big), axis=1,
                         keepdims=True)
        else:
            am = jnp.argmin(d2, axis=1).astype(jnp.int32)[:, None]
        idx_ref[0, :, k:k + 1] = am
        hit = iota == am
        if want_rel:
            rel_ref[0, :, k:k + 1] = jnp.sum(
                jnp.where(hit, diff0, 0.0), axis=1, keepdims=True)
        d2 = jnp.where(hit, jnp.inf, d2)


def _knn(qpts, kpts_t, K, want_rel, tq):
    B, Nq, D = qpts.shape
    N = kpts_t.shape[2]
    grid = (B, Nq // tq)
    out_shapes = [jax.ShapeDtypeStruct((B, Nq, K), jnp.int32)]
    out_specs = [pl.BlockSpec((1, tq, K), lambda b, i: (b, i, 0))]
    if want_rel:
        out_shapes.append(jax.ShapeDtypeStruct((B, Nq, K), jnp.float32))
        out_specs.append(pl.BlockSpec((1, tq, K), lambda b, i: (b, i, 0)))
    body = functools.partial(_knn_body, K=K, D=D, N=N, want_rel=want_rel)
    if not want_rel:
        body2 = lambda q, kk, i: body(q, kk, i, None)
    else:
        body2 = body
    res = pl.pallas_call(
        body2,
        grid=grid,
        in_specs=[
            pl.BlockSpec((1, tq, D), lambda b, i: (b, i, 0)),
            pl.BlockSpec((1, D, N), lambda b, i: (b, 0, 0)),
        ],
        out_specs=out_specs,
        out_shape=out_shapes,
    )(qpts, kpts_t)
    return res if want_rel else (res[0], None)


_NBUF = 4


@functools.lru_cache(maxsize=None)
def _gmean_sc_v2(NQ, K, C, per_qk, CQ):
    nq_w = NQ // _NW
    steps = nq_w // CQ
    assert CQ * K <= 128 and steps % _NBUF == 0
    nj = C // 16
    nk_rows = CQ * K
    mesh = plsc.VectorSubcoreMesh(core_axis_name="c", subcore_axis_name="s")
    add_rows = nk_rows if per_qk else nq_w

    @functools.partial(
        pl.kernel, mesh=mesh,
        out_type=jax.ShapeDtypeStruct((NQ, C), jnp.float32),
        scratch_types=(
            [pltpu.VMEM((nq_w * K,), jnp.int32)]
            + [pltpu.VMEM((nk_rows, 128), jnp.float32)] * _NBUF
            + [pltpu.VMEM((add_rows, C), jnp.float32)]
            + [pltpu.VMEM((nk_rows if per_qk else 1, C), jnp.float32)] * (_NBUF - 1)
            + [pltpu.VMEM((nq_w, C), jnp.float32)]
            + [pltpu.SemaphoreType.DMA] * _NBUF
        ))
    def kfun(z_hbm, gidx_hbm, add_hbm, out_hbm, idx_all, *sc):
        rows_b = sc[0:_NBUF]
        add_b = sc[_NBUF:2 * _NBUF]
        out_v = sc[2 * _NBUF]
        sem_b = sc[2 * _NBUF + 1:]
        wid = lax.axis_index("s") * 2 + lax.axis_index("c")
        qbase = wid * nq_w

        pltpu.sync_copy(gidx_hbm.at[pl.ds(qbase * K, nq_w * K)], idx_all)
        if not per_qk:
            pltpu.sync_copy(add_hbm.at[pl.ds(qbase, nq_w)], add_b[0])

        def fire(c, b):
            pltpu.async_copy(
                z_hbm.at[idx_all.at[pl.ds(c * nk_rows, nk_rows)]],
                rows_b[b], sem_b[b])
            if per_qk:
                pltpu.async_copy(
                    add_hbm.at[pl.ds((qbase + c * CQ) * K, nk_rows)],
                    add_b[b], sem_b[b])

        def drain(b):
            pltpu.make_async_copy(
                z_hbm.at[pl.ds(0, nk_rows)], rows_b[b], sem_b[b]).wait()
            if per_qk:
                pltpu.make_async_copy(
                    add_hbm.at[pl.ds(0, nk_rows)], add_b[b], sem_b[b]).wait()

        def compute(c, b):
            rows = rows_b[b]
            adds = add_b[b]
            qloc = c * CQ
            inv_k = jnp.float32(1.0 / K)
            for q in range(CQ):
                for j in range(nj):
                    sl = pl.ds(j * 16, 16)
                    acc = jnp.zeros((16,), jnp.float32)
                    if not per_qk:
                        a = add_b[0][qloc + q, sl]
                    for k in range(K):
                        r = q * K + k
                        h = rows[r, sl] + (adds[r, sl] if per_qk else a)
                        acc = acc + jnp.maximum(h, 0.0)
                    out_v[qloc + q, sl] = acc * inv_k

        for b in range(_NBUF):
            fire(jnp.int32(b), b)

        def step(s, carry):
            c0 = _NBUF * s
            for b in range(_NBUF):
                drain(b)
                compute(c0 + b, b)
                fire(jnp.minimum(c0 + b + _NBUF, steps - 1), b)
            return carry

        lax.fori_loop(0, steps // _NBUF, step, 0)
        for b in range(_NBUF):
            drain(b)
        pltpu.sync_copy(out_v, out_hbm.at[pl.ds(qbase, nq_w)])

    return kfun


@functools.lru_cache(maxsize=None)
def _gmean_sc(NQ, K, C, per_qk, CQ):
    nq_w = NQ // _NW
    steps = nq_w // CQ
    assert CQ * K <= 128 and nq_w % CQ == 0
    nj = C // 16
    mesh = plsc.VectorSubcoreMesh(core_axis_name="c", subcore_axis_name="s")

    @functools.partial(
        pl.kernel, mesh=mesh,
        out_type=jax.ShapeDtypeStruct((NQ, C), jnp.float32),
        scratch_types=[
            pltpu.VMEM((CQ * K,), jnp.int32),
            pltpu.VMEM((CQ * K, 128), jnp.float32),
            pltpu.VMEM((CQ * K if per_qk else CQ, C), jnp.float32),
            pltpu.VMEM((CQ, C), jnp.float32),
            pltpu.SemaphoreType.DMA,
        ])
    def kfun(z_hbm, gidx_hbm, add_hbm, out_hbm, idx_v, rows_v, add_v, out_v, sem):
        wid = lax.axis_index("s") * 2 + lax.axis_index("c")
        qbase = wid * nq_w

        def step(s, carry):
            qb = qbase + s * CQ
            pltpu.sync_copy(gidx_hbm.at[pl.ds(qb * K, CQ * K)], idx_v)
            if per_qk:
                pltpu.sync_copy(add_hbm.at[pl.ds(qb * K, CQ * K)], add_v)
            else:
                pltpu.sync_copy(add_hbm.at[pl.ds(qb, CQ)], add_v)
            pltpu.async_copy(z_hbm.at[idx_v], rows_v, sem).wait()
            inv_k = jnp.float32(1.0 / K)
            for q in range(CQ):
                for j in range(nj):
                    sl = pl.ds(j * 16, 16)
                    acc = jnp.zeros((16,), jnp.float32)
                    if not per_qk:
                        a = add_v[q, sl]
                    for k in range(K):
                        r = q * K + k
                        h = rows_v[r, sl] + (add_v[r, sl] if per_qk else a)
                        acc = acc + jnp.maximum(h, 0.0)
                    out_v[q, sl] = acc * inv_k
            pltpu.sync_copy(out_v, out_hbm.at[pl.ds(qb, CQ)])
            return carry

        lax.fori_loop(0, steps, step, 0)

    return kfun


def _gmean(z, idx, add_pq=None, add_pqk=None):
    B, NR, C = z.shape
    _, NQ_b, K = idx.shape
    NQ = B * NQ_b
    gidx = (idx + (jnp.arange(B, dtype=jnp.int32) * NR)[:, None, None])
    gidx = gidx.reshape(NQ * K)
    zf = z.reshape(B * NR, C)
    if C < 128:
        zf = jnp.pad(zf, ((0, 0), (0, 128 - C)))
    if add_pqk is not None:
        add = add_pqk.reshape(NQ * K, C)
        per_qk = True
    else:
        add = add_pq.reshape(NQ, C)
        per_qk = False
    CQ = (64 if C <= 64 else 32) // K
    out = _gmean_sc_v2(NQ, K, C, per_qk, CQ)(zf, gidx, add)
    return out.reshape(B, NQ_b, C)


def kernel(data, ids, space_pts, time_pts, query_pts, te_w, te_phase,
           s0_Wf, s0_Wr, s0_b, t0_Wf, t0_Wr, t0_b, c0_W1, c0_b1, c0_W2, c0_b2,
           s1_Wf, s1_Wr, s1_b, t1_Wf, t1_Wr, t1_b, c1_W1, c1_b1, c1_W2, c1_b2,
           tg_Wf, tg_Wr, tg_b):
    B, N, F = data.shape
    Q = query_pts.shape[1]

    sp_t = jnp.transpose(space_pts, (0, 2, 1))
    tp_t = jnp.transpose(time_pts, (0, 2, 1))

    idx_s, _ = _knn(space_pts, sp_t, NEIGHBORS, False, 256)
    idx_t, rel_t = _knn(time_pts, tp_t, TIMESTEPS, True, 256)
    idx_q, rel_q = _knn(query_pts, tp_t, TIMESTEPS, True, 512)

    E_t = jnp.cos(rel_t[..., None] * te_w + te_phase)
    E_q = jnp.cos(rel_q[..., None] * te_w + te_phase)

    x = data
    for (sWf, sWr, sb, tWf, tWr, tb, cW1, cb1, cW2, cb2) in [
            (s0_Wf, s0_Wr, s0_b, t0_Wf, t0_Wr, t0_b, c0_W1, c0_b1, c0_W2, c0_b2),
            (s1_Wf, s1_Wr, s1_b, t1_Wf, t1_Wr, t1_b, c1_W1, c1_b1, c1_W2, c1_b2)]:
        u = space_pts @ sWr
        z_s = x @ sWf - u
        a_s = u + sb
        snei = _gmean(z_s, idx_s, add_pq=a_s)
        z_t = x @ tWf[:F] + snei @ tWf[F:]
        relc = E_t @ tWr + tb
        tnei = _gmean(z_t, idx_t, add_pqk=relc)
        x = jnp.maximum(
            x @ cW1[:F] + snei @ cW1[F:F + 64] + tnei @ cW1[F + 64:] + cb1,
            0.0) @ cW2 + cb2
    z_g = x @ tg_Wf
    relc_q = E_q @ tg_Wr + tg_b
    return _gmean(z_g, idx_q, add_pqk=relc_q)
